# baseline (speedup 1.0000x reference)
import functools

import jax
import jax.numpy as jnp
from jax import lax
from jax.experimental import pallas as pl
from jax.experimental.pallas import tpu as pltpu
from jax.experimental.pallas import tpu_sc as plsc

N = 10000
E = 320000
DIN = 128
DOUT = 64
K = 2
A = 1

NC, NS, L = 2, 16, 16
NW = NC * NS
NPT = 320
NPAD = NW * NPT
C = 128
NF = 2 + A + K
KD = K * DOUT
KA = K * (A + 1)
ARW = 16


def _f16(v, dtype=jnp.int32):
    return jnp.full((L,), v, dtype)


def _scalar(vec):
    return jnp.max(vec)


def _dense_body(x_ref, wcat_ref, cl_ref, cr_ref, wx_ref, al_ref, ar_ref):
    wx = jnp.dot(x_ref[...], wcat_ref[...], preferred_element_type=jnp.float32)
    wx_ref[...] = wx
    al_ref[...] = jnp.dot(wx, cl_ref[...], preferred_element_type=jnp.float32)
    ar_ref[...] = jnp.dot(wx, cr_ref[...], preferred_element_type=jnp.float32)


def _offs_body(rows_ref, offs_ref):
    blk = rows_ref[...]
    lane = lax.broadcasted_iota(jnp.int32, (1, 128), 1)
    acc = jnp.zeros((1, 128), jnp.int32)
    for w in range(NW + 1):
        s = jnp.sum((blk < w * NPT).astype(jnp.int32))
        acc = acc + jnp.where(lane == w, s, 0)
    offs_ref[...] = acc


ACC_W = K * DOUT + K * L


def _sc_body(wx_hbm, ed_hbm, al_hbm, ar_hbm, offs_hbm, out_hbm,
             arg0_v, arg1_v, al_v, offs_v, ed0_v, ed1_v, wxg0_v, wxg1_v,
             e_v, acc_v, sw0, sw1, sa0, sa1):
    wid = lax.axis_index("s") * NC + lax.axis_index("c")
    n0 = wid * NPT
    bufs = ((ed0_v, wxg0_v, arg0_v, sw0, sa0),
            (ed1_v, wxg1_v, arg1_v, sw1, sa1))

    pltpu.sync_copy(al_hbm.at[pl.ds(n0, NPT)], al_v)
    pltpu.sync_copy(offs_hbm, offs_v)

    zf = jnp.zeros((L,), jnp.float32)

    @plsc.parallel_loop(0, NPT, unroll=4)
    def _(i):
        for v in range(ACC_W // L):
            acc_v[i, pl.ds(v * L, L)] = zf

    start = _scalar(plsc.load_gather(offs_v, [_f16(0), _f16(wid)]))
    end = _scalar(plsc.load_gather(offs_v, [_f16(0), _f16(wid + 1)]))
    astart = (start // C) * C
    nch = (end - astart + C - 1) // C

    def _fetch(g, ed_v, wxg_v, arg_v, s_wx, s_ar):
        s = astart + g * C
        pltpu.sync_copy(ed_hbm.at[:, pl.ds(s, C)], ed_v.at[:, :C])
        pltpu.async_copy(wx_hbm.at[ed_v.at[1, :C]], wxg_v, s_wx)
        pltpu.async_copy(ar_hbm.at[ed_v.at[1, :C]], arg_v, s_ar)

    def _compute(g, ed_v, wxg_v, arg_v, s_wx, s_ar):
        s = astart + g * C
        pltpu.make_async_copy(wx_hbm.at[ed_v.at[1, :C]], wxg_v, s_wx).wait()
        pltpu.make_async_copy(ar_hbm.at[ed_v.at[1, :C]], arg_v, s_ar).wait()

        for j in range(C // L):
            b = j * L
            rows_v = ed_v[0, pl.ds(b, L)]
            cols_v = ed_v[1, pl.ds(b, L)]
            li = jnp.clip(rows_v - n0, 0, NPT - 1)
            bi = lax.broadcasted_iota(jnp.int32, (L,), 0) + b
            aL = [plsc.load_gather(al_v, [li, _f16(t)]) for t in range(KA)]
            aR = [plsc.load_gather(arg_v, [bi, _f16(t)]) for t in range(KA)]
            for k in range(K):
                t_k = zf
                for i in range(A):
                    av = plsc.bitcast(ed_v[2 + i, pl.ds(b, L)], jnp.float32)
                    t_k = t_k + av * (aL[k * (A + 1) + i] + aR[k * (A + 1) + i])
                sv = plsc.bitcast(ed_v[2 + A + k, pl.ds(b, L)], jnp.float32)
                t_k = t_k + sv * (aL[k * (A + 1) + A] + aR[k * (A + 1) + A])
                e_v[k, pl.ds(b, L)] = jnp.exp(t_k)

        i0 = jnp.maximum(0, start - s)
        vn = jnp.minimum(C, end - s)

        @plsc.parallel_loop(i0, vn, unroll=8)
        def _(i):
            lr = ed_v[0, pl.ds(i, L)][0] - n0
            es = [plsc.load_gather(e_v, [_f16(k), _f16(i)]) for k in range(K)]
            for k in range(K):
                for v in range(DOUT // L):
                    col = k * DOUT + v * L
                    wv = wxg_v[i, pl.ds(col, L)]
                    plsc.addupdate(acc_v.at[lr, pl.ds(col, L)], es[k] * wv)
            for k in range(K):
                plsc.addupdate(acc_v.at[lr, pl.ds(K * DOUT + k * L, L)], es[k])

    @pl.when(nch > 0)
    def _():
        _fetch(0, *bufs[0])

    def outer(j, _):
        for b in range(2):
            g = 2 * j + b

            @pl.when(g + 1 < nch)
            def _():
                _fetch(g + 1, *bufs[(b + 1) % 2])

            @pl.when(g < nch)
            def _():
                _compute(g, *bufs[b])
        return 0

    lax.fori_loop(0, (nch + 1) // 2, outer, 0)

    @plsc.parallel_loop(0, NPT, unroll=2)
    def _(n):
        dens = [acc_v[n, pl.ds(K * DOUT + k * L, L)] for k in range(K)]
        invs = [jnp.where(d > 0, 1.0 / d, zf) for d in dens]
        for v in range(DOUT // L):
            o = zf
            for k in range(K):
                o = o + acc_v[n, pl.ds(k * DOUT + v * L, L)] * invs[k]
            o = jnp.where(o > 0, o, jnp.exp(o) - 1.0)
            acc_v[n, pl.ds(v * L, L)] = o

    pltpu.sync_copy(acc_v.at[:, :DOUT], out_hbm.at[pl.ds(n0, NPT)])


def kernel(x, edge_index, support_values, atten_values, W_t, W_l, W_r):
    rows = edge_index[:, 0].astype(jnp.int32)
    cols = edge_index[:, 1].astype(jnp.int32)

    Wcat = jnp.concatenate([W_t[k] for k in range(K)], axis=1)
    CL = jnp.zeros((KD, KA), jnp.float32)
    CR = jnp.zeros((KD, ARW), jnp.float32)
    for k in range(K):
        for i in range(A + 1):
            CL = CL.at[k * DOUT:(k + 1) * DOUT, k * (A + 1) + i].set(W_l[k, i, :, 0])
            CR = CR.at[k * DOUT:(k + 1) * DOUT, k * (A + 1) + i].set(W_r[k, i, :, 0])

    GB = 10
    WX, AL, AR = pl.pallas_call(
        _dense_body,
        grid=(GB,),
        in_specs=[pl.BlockSpec((N // GB, DIN), lambda i: (i, 0)),
                  pl.BlockSpec((DIN, KD), lambda i: (0, 0)),
                  pl.BlockSpec((KD, KA), lambda i: (0, 0)),
                  pl.BlockSpec((KD, ARW), lambda i: (0, 0))],
        out_specs=[pl.BlockSpec((N // GB, KD), lambda i: (i, 0)),
                   pl.BlockSpec((N // GB, KA), lambda i: (i, 0)),
                   pl.BlockSpec((N // GB, ARW), lambda i: (i, 0))],
        out_shape=[jax.ShapeDtypeStruct((N, KD), jnp.float32),
                   jax.ShapeDtypeStruct((N, KA), jnp.float32),
                   jax.ShapeDtypeStruct((N, ARW), jnp.float32)],
    )(x, Wcat, CL, CR)

    offs = pl.pallas_call(
        _offs_body,
        in_specs=[pl.BlockSpec((E // 128, 128), lambda: (0, 0))],
        out_specs=pl.BlockSpec((1, 128), lambda: (0, 0)),
        out_shape=jax.ShapeDtypeStruct((1, 128), jnp.int32),
    )(rows.reshape(E // 128, 128))

    fields = [rows, cols]
    for i in range(A):
        fields.append(lax.bitcast_convert_type(atten_values[i], jnp.int32))
    for k in range(K):
        fields.append(lax.bitcast_convert_type(support_values[k], jnp.int32))
    ed = jnp.stack(fields, axis=0)
    ed = jnp.concatenate([ed, jnp.zeros((NF, C), jnp.int32)], axis=1)

    AL_pad = jnp.concatenate([AL, jnp.zeros((NPAD - N, KA), jnp.float32)])

    mesh = plsc.VectorSubcoreMesh(core_axis_name="c", subcore_axis_name="s")
    out_pad = pl.kernel(
        _sc_body,
        out_type=jax.ShapeDtypeStruct((NPAD, DOUT), jnp.float32),
        mesh=mesh,
        compiler_params=pltpu.CompilerParams(needs_layout_passes=False,
                                             use_tc_tiling_on_sc=False),
        scratch_types=[
            pltpu.VMEM((C, ARW), jnp.float32),
            pltpu.VMEM((C, ARW), jnp.float32),
            pltpu.VMEM((NPT, KA), jnp.float32),
            pltpu.VMEM((1, 128), jnp.int32),
            pltpu.VMEM((NF, C + L), jnp.int32),
            pltpu.VMEM((NF, C + L), jnp.int32),
            pltpu.VMEM((C, KD), jnp.float32),
            pltpu.VMEM((C, KD), jnp.float32),
            pltpu.VMEM((K, C), jnp.float32),
            pltpu.VMEM((NPT, ACC_W), jnp.float32),
            pltpu.SemaphoreType.DMA,
            pltpu.SemaphoreType.DMA,
            pltpu.SemaphoreType.DMA,
            pltpu.SemaphoreType.DMA,
        ],
    )(WX, ed, AL_pad, AR, offs)

    return out_pad[:N]


# speedup vs baseline: 1.0133x; 1.0133x over previous
import functools

import jax
import jax.numpy as jnp
from jax import lax
from jax.experimental import pallas as pl
from jax.experimental.pallas import tpu as pltpu
from jax.experimental.pallas import tpu_sc as plsc

N = 10000
E = 320000
DIN = 128
DOUT = 64
K = 2
A = 1

NC, NS, L = 2, 16, 16
NW = NC * NS
NPT = 320
NPAD = NW * NPT
C = 128
NF = 2 + A + K
KD = K * DOUT
KA = K * (A + 1)
ARW = 16


def _f16(v, dtype=jnp.int32):
    return jnp.full((L,), v, dtype)


def _scalar(vec):
    return jnp.max(vec)


def _dense_body(x_ref, wcat_ref, cl_ref, cr_ref, wx_ref, al_ref, ar_ref):
    wx = jnp.dot(x_ref[...], wcat_ref[...], preferred_element_type=jnp.float32)
    wx_ref[...] = wx
    al_ref[...] = jnp.dot(wx, cl_ref[...], preferred_element_type=jnp.float32)
    ar_ref[...] = jnp.dot(wx, cr_ref[...], preferred_element_type=jnp.float32)


def _offs_body(rows_ref, offs_ref):
    blk = rows_ref[...]
    lane = lax.broadcasted_iota(jnp.int32, (1, 128), 1)
    acc = jnp.zeros((1, 128), jnp.int32)
    for w in range(NW + 1):
        s = jnp.sum((blk < w * NPT).astype(jnp.int32))
        acc = acc + jnp.where(lane == w, s, 0)
    offs_ref[...] = acc


ACC_W = K * DOUT + L


def _sc_body(wx_hbm, ed_hbm, al_hbm, ar_hbm, offs_hbm, out_hbm,
             arg0_v, arg1_v, al_v, offs_v, ed0_v, ed1_v, wxg0_v, wxg1_v,
             e_v, acc_v, sw0, sw1, sa0, sa1):
    wid = lax.axis_index("s") * NC + lax.axis_index("c")
    n0 = wid * NPT
    bufs = ((ed0_v, wxg0_v, arg0_v, sw0, sa0),
            (ed1_v, wxg1_v, arg1_v, sw1, sa1))

    pltpu.sync_copy(al_hbm.at[pl.ds(n0, NPT)], al_v)
    pltpu.sync_copy(offs_hbm, offs_v)

    zf = jnp.zeros((L,), jnp.float32)

    @plsc.parallel_loop(0, NPT, unroll=4)
    def _(i):
        for v in range(ACC_W // L):
            acc_v[i, pl.ds(v * L, L)] = zf

    start = _scalar(plsc.load_gather(offs_v, [_f16(0), _f16(wid)]))
    end = _scalar(plsc.load_gather(offs_v, [_f16(0), _f16(wid + 1)]))
    astart = (start // C) * C
    nch = (end - astart + C - 1) // C

    def _fetch(g, ed_v, wxg_v, arg_v, s_wx, s_ar):
        s = astart + g * C
        pltpu.sync_copy(ed_hbm.at[:, pl.ds(s, C)], ed_v.at[:, :C])
        pltpu.async_copy(wx_hbm.at[ed_v.at[1, :C]], wxg_v, s_wx)
        pltpu.async_copy(ar_hbm.at[ed_v.at[1, :C]], arg_v, s_ar)

    def _compute(g, ed_v, wxg_v, arg_v, s_wx, s_ar):
        s = astart + g * C
        pltpu.make_async_copy(wx_hbm.at[ed_v.at[1, :C]], wxg_v, s_wx).wait()
        pltpu.make_async_copy(ar_hbm.at[ed_v.at[1, :C]], arg_v, s_ar).wait()

        for j in range(C // L):
            b = j * L
            rows_v = ed_v[0, pl.ds(b, L)]
            cols_v = ed_v[1, pl.ds(b, L)]
            li = jnp.clip(rows_v - n0, 0, NPT - 1)
            bi = lax.broadcasted_iota(jnp.int32, (L,), 0) + b
            aL = [plsc.load_gather(al_v, [li, _f16(t)]) for t in range(KA)]
            aR = [plsc.load_gather(arg_v, [bi, _f16(t)]) for t in range(KA)]
            for k in range(K):
                t_k = zf
                for i in range(A):
                    av = plsc.bitcast(ed_v[2 + i, pl.ds(b, L)], jnp.float32)
                    t_k = t_k + av * (aL[k * (A + 1) + i] + aR[k * (A + 1) + i])
                sv = plsc.bitcast(ed_v[2 + A + k, pl.ds(b, L)], jnp.float32)
                t_k = t_k + sv * (aL[k * (A + 1) + A] + aR[k * (A + 1) + A])
                e_v[k, pl.ds(b, L)] = jnp.exp(t_k)

        i0 = jnp.maximum(0, start - s)
        vn = jnp.minimum(C, end - s)

        lane = lax.broadcasted_iota(jnp.int32, (L,), 0)

        @plsc.parallel_loop(i0, vn, unroll=4)
        def _(i):
            lr = ed_v[0, pl.ds(i, L)][0] - n0
            es = [plsc.load_gather(e_v, [_f16(k), _f16(i)]) for k in range(K)]
            for k in range(K):
                for v in range(DOUT // L):
                    col = k * DOUT + v * L
                    wv = wxg_v[i, pl.ds(col, L)]
                    plsc.addupdate(acc_v.at[lr, pl.ds(col, L)], es[k] * wv)
            dcomb = jnp.where(lane < L // 2, es[0], es[1])
            plsc.addupdate(acc_v.at[lr, pl.ds(K * DOUT, L)], dcomb)

    @pl.when(nch > 0)
    def _():
        _fetch(0, *bufs[0])

    def outer(j, _):
        for b in range(2):
            g = 2 * j + b

            @pl.when(g + 1 < nch)
            def _():
                _fetch(g + 1, *bufs[(b + 1) % 2])

            @pl.when(g < nch)
            def _():
                _compute(g, *bufs[b])
        return 0

    lax.fori_loop(0, (nch + 1) // 2, outer, 0)

    @plsc.parallel_loop(0, NPT, unroll=2)
    def _(n):
        dslot = acc_v[n, pl.ds(K * DOUT, L)]
        dsafe = jnp.where(dslot > 0, dslot, jnp.full((L,), 1.0, jnp.float32))
        dinv = 1.0 / dsafe
        invs = [jnp.full((L,), dinv[k * (L // K)], jnp.float32)
                for k in range(K)]
        for v in range(DOUT // L):
            o = zf
            for k in range(K):
                o = o + acc_v[n, pl.ds(k * DOUT + v * L, L)] * invs[k]
            o = jnp.where(o > 0, o, jnp.exp(o) - 1.0)
            acc_v[n, pl.ds(v * L, L)] = o

    pltpu.sync_copy(acc_v.at[:, :DOUT], out_hbm.at[pl.ds(n0, NPT)])


def kernel(x, edge_index, support_values, atten_values, W_t, W_l, W_r):
    rows = edge_index[:, 0].astype(jnp.int32)
    cols = edge_index[:, 1].astype(jnp.int32)

    Wcat = jnp.concatenate([W_t[k] for k in range(K)], axis=1)
    CL = jnp.zeros((KD, KA), jnp.float32)
    CR = jnp.zeros((KD, ARW), jnp.float32)
    for k in range(K):
        for i in range(A + 1):
            CL = CL.at[k * DOUT:(k + 1) * DOUT, k * (A + 1) + i].set(W_l[k, i, :, 0])
            CR = CR.at[k * DOUT:(k + 1) * DOUT, k * (A + 1) + i].set(W_r[k, i, :, 0])

    GB = 10
    WX, AL, AR = pl.pallas_call(
        _dense_body,
        grid=(GB,),
        in_specs=[pl.BlockSpec((N // GB, DIN), lambda i: (i, 0)),
                  pl.BlockSpec((DIN, KD), lambda i: (0, 0)),
                  pl.BlockSpec((KD, KA), lambda i: (0, 0)),
                  pl.BlockSpec((KD, ARW), lambda i: (0, 0))],
        out_specs=[pl.BlockSpec((N // GB, KD), lambda i: (i, 0)),
                   pl.BlockSpec((N // GB, KA), lambda i: (i, 0)),
                   pl.BlockSpec((N // GB, ARW), lambda i: (i, 0))],
        out_shape=[jax.ShapeDtypeStruct((N, KD), jnp.float32),
                   jax.ShapeDtypeStruct((N, KA), jnp.float32),
                   jax.ShapeDtypeStruct((N, ARW), jnp.float32)],
    )(x, Wcat, CL, CR)

    offs = pl.pallas_call(
        _offs_body,
        in_specs=[pl.BlockSpec((E // 128, 128), lambda: (0, 0))],
        out_specs=pl.BlockSpec((1, 128), lambda: (0, 0)),
        out_shape=jax.ShapeDtypeStruct((1, 128), jnp.int32),
    )(rows.reshape(E // 128, 128))

    fields = [rows, cols]
    for i in range(A):
        fields.append(lax.bitcast_convert_type(atten_values[i], jnp.int32))
    for k in range(K):
        fields.append(lax.bitcast_convert_type(support_values[k], jnp.int32))
    ed = jnp.stack(fields, axis=0)
    ed = jnp.concatenate([ed, jnp.zeros((NF, C), jnp.int32)], axis=1)

    AL_pad = jnp.concatenate([AL, jnp.zeros((NPAD - N, KA), jnp.float32)])

    mesh = plsc.VectorSubcoreMesh(core_axis_name="c", subcore_axis_name="s")
    out_pad = pl.kernel(
        _sc_body,
        out_type=jax.ShapeDtypeStruct((NPAD, DOUT), jnp.float32),
        mesh=mesh,
        compiler_params=pltpu.CompilerParams(needs_layout_passes=False,
                                             use_tc_tiling_on_sc=False),
        scratch_types=[
            pltpu.VMEM((C, ARW), jnp.float32),
            pltpu.VMEM((C, ARW), jnp.float32),
            pltpu.VMEM((NPT, KA), jnp.float32),
            pltpu.VMEM((1, 128), jnp.int32),
            pltpu.VMEM((NF, C + L), jnp.int32),
            pltpu.VMEM((NF, C + L), jnp.int32),
            pltpu.VMEM((C, KD), jnp.float32),
            pltpu.VMEM((C, KD), jnp.float32),
            pltpu.VMEM((K, C), jnp.float32),
            pltpu.VMEM((NPT, ACC_W), jnp.float32),
            pltpu.SemaphoreType.DMA,
            pltpu.SemaphoreType.DMA,
            pltpu.SemaphoreType.DMA,
            pltpu.SemaphoreType.DMA,
        ],
    )(WX, ed, AL_pad, AR, offs)

    return out_pad[:N]


# speedup vs baseline: 1.1924x; 1.1767x over previous
import functools

import jax
import jax.numpy as jnp
from jax import lax
from jax.experimental import pallas as pl
from jax.experimental.pallas import tpu as pltpu
from jax.experimental.pallas import tpu_sc as plsc

N = 10000
E = 320000
DIN = 128
DOUT = 64
K = 2
A = 1

NC, NS, L = 2, 16, 16
NW = NC * NS
NPT = 320
NPAD = NW * NPT
C = 128
NF = 2 + A + K
KD = K * DOUT
KA = K * (A + 1)
ARW = 16


def _f16(v, dtype=jnp.int32):
    return jnp.full((L,), v, dtype)


def _scalar(vec):
    return jnp.max(vec)


def _dense_body(x_ref, wcat_ref, wcatp_ref, cl_ref, cr_ref,
                wxb_ref, al_ref, ar_ref):
    wx = jnp.dot(x_ref[...], wcat_ref[...], preferred_element_type=jnp.float32)
    wxp = jnp.dot(x_ref[...], wcatp_ref[...],
                  preferred_element_type=jnp.float32)
    wxb_ref[...] = wxp.astype(jnp.bfloat16)
    al_ref[...] = jnp.dot(wx, cl_ref[...], preferred_element_type=jnp.float32)
    ar_ref[...] = jnp.dot(wx, cr_ref[...], preferred_element_type=jnp.float32)


def _offs_body(rows_ref, offs_ref):
    blk = rows_ref[...]
    lane = lax.broadcasted_iota(jnp.int32, (1, 128), 1)
    acc = jnp.zeros((1, 128), jnp.int32)
    for w in range(NW + 1):
        s = jnp.sum((blk < w * NPT).astype(jnp.int32))
        acc = acc + jnp.where(lane == w, s, 0)
    offs_ref[...] = acc


ACC_W = K * DOUT + L


NB = 4


def _sc_body(wx_hbm, ed_hbm, al_hbm, ar_hbm, offs_hbm, out_hbm,
             arg_vs, al_v, offs_v, ed_vs, wxg_vs, e_v, acc_v,
             sed_s, swx_s, sar_s):
    wid = lax.axis_index("s") * NC + lax.axis_index("c")
    n0 = wid * NPT
    bufs = tuple((ed_vs[b], wxg_vs[b], arg_vs[b], sed_s[b], swx_s[b],
                  sar_s[b]) for b in range(NB))

    pltpu.sync_copy(al_hbm.at[pl.ds(n0, NPT)], al_v)
    pltpu.sync_copy(offs_hbm, offs_v)

    zf = jnp.zeros((L,), jnp.float32)

    @plsc.parallel_loop(0, NPT, unroll=4)
    def _(i):
        for v in range(ACC_W // L):
            acc_v[i, pl.ds(v * L, L)] = zf

    start = _scalar(plsc.load_gather(offs_v, [_f16(0), _f16(wid)]))
    end = _scalar(plsc.load_gather(offs_v, [_f16(0), _f16(wid + 1)]))
    astart = (start // C) * C
    nch = (end - astart + C - 1) // C

    def _fetch_ed(g, ed_v, wxg_v, arg_v, s_ed, s_wx, s_ar):
        s = astart + g * C
        pltpu.async_copy(ed_hbm.at[:, pl.ds(s, C)], ed_v.at[:, :C], s_ed)

    def _fire(g, ed_v, wxg_v, arg_v, s_ed, s_wx, s_ar):
        s = astart + g * C
        pltpu.make_async_copy(ed_hbm.at[:, pl.ds(s, C)], ed_v.at[:, :C],
                              s_ed).wait()
        pltpu.async_copy(wx_hbm.at[ed_v.at[1, :C]], wxg_v, s_wx)
        pltpu.async_copy(ar_hbm.at[ed_v.at[1, :C]], arg_v, s_ar)

    def _compute(g, ed_v, wxg_v, arg_v, s_ed, s_wx, s_ar):
        s = astart + g * C
        pltpu.make_async_copy(wx_hbm.at[ed_v.at[1, :C]], wxg_v, s_wx).wait()
        pltpu.make_async_copy(ar_hbm.at[ed_v.at[1, :C]], arg_v, s_ar).wait()

        for j in range(C // L):
            b = j * L
            rows_v = ed_v[0, pl.ds(b, L)]
            cols_v = ed_v[1, pl.ds(b, L)]
            li = jnp.clip(rows_v - n0, 0, NPT - 1)
            bi = lax.broadcasted_iota(jnp.int32, (L,), 0) + b
            aL = [plsc.load_gather(al_v, [li, _f16(t)]) for t in range(KA)]
            aR = [plsc.load_gather(arg_v, [bi, _f16(t)]) for t in range(KA)]
            for k in range(K):
                t_k = zf
                for i in range(A):
                    av = plsc.bitcast(ed_v[2 + i, pl.ds(b, L)], jnp.float32)
                    t_k = t_k + av * (aL[k * (A + 1) + i] + aR[k * (A + 1) + i])
                sv = plsc.bitcast(ed_v[2 + A + k, pl.ds(b, L)], jnp.float32)
                t_k = t_k + sv * (aL[k * (A + 1) + A] + aR[k * (A + 1) + A])
                e_v[k, pl.ds(b, L)] = jnp.exp(t_k)

        i0 = jnp.maximum(0, start - s)
        vn = jnp.minimum(C, end - s)

        lane = lax.broadcasted_iota(jnp.int32, (L,), 0)

        @plsc.parallel_loop(i0, vn, unroll=4)
        def _(i):
            lr = ed_v[0, pl.ds(i, L)][0] - n0
            es = [plsc.load_gather(e_v, [_f16(k), _f16(i)]) for k in range(K)]
            for k in range(K):
                for h in range(DOUT // (2 * L)):
                    col = k * DOUT + h * 2 * L
                    wv2 = wxg_v[i, pl.ds(col, 2 * L)]
                    wa, wb = plsc.unpack(wv2, format=plsc.PackFormat.INTERLEAVED)
                    plsc.addupdate(acc_v.at[lr, pl.ds(col, L)], es[k] * wa)
                    plsc.addupdate(acc_v.at[lr, pl.ds(col + L, L)], es[k] * wb)
            dcomb = jnp.where(lane < L // 2, es[0], es[1])
            plsc.addupdate(acc_v.at[lr, pl.ds(K * DOUT, L)], dcomb)

    for p in range(NB - 1):
        @pl.when(p < nch)
        def _(p=p):
            _fetch_ed(p, *bufs[p])
    for p in range(NB - 2):
        @pl.when(p < nch)
        def _(p=p):
            _fire(p, *bufs[p])

    def outer(j, _):
        for b in range(NB):
            g = NB * j + b

            @pl.when(g + NB - 1 < nch)
            def _():
                _fetch_ed(g + NB - 1, *bufs[(b + NB - 1) % NB])

            @pl.when(g + NB - 2 < nch)
            def _():
                _fire(g + NB - 2, *bufs[(b + NB - 2) % NB])

            @pl.when(g < nch)
            def _():
                _compute(g, *bufs[b])
        return 0

    lax.fori_loop(0, (nch + NB - 1) // NB, outer, 0)

    @plsc.parallel_loop(0, NPT, unroll=2)
    def _(n):
        dslot = acc_v[n, pl.ds(K * DOUT, L)]
        dsafe = jnp.where(dslot > 0, dslot, jnp.full((L,), 1.0, jnp.float32))
        dinv = 1.0 / dsafe
        invs = [jnp.full((L,), dinv[k * (L // K)], jnp.float32)
                for k in range(K)]
        for v in range(DOUT // L):
            o = zf
            for k in range(K):
                o = o + acc_v[n, pl.ds(k * DOUT + v * L, L)] * invs[k]
            o = jnp.where(o > 0, o, jnp.exp(o) - 1.0)
            acc_v[n, pl.ds(v * L, L)] = o

    pltpu.sync_copy(acc_v.at[:, :DOUT], out_hbm.at[pl.ds(n0, NPT)])


def kernel(x, edge_index, support_values, atten_values, W_t, W_l, W_r):
    rows = edge_index[:, 0].astype(jnp.int32)
    cols = edge_index[:, 1].astype(jnp.int32)

    Wcat = jnp.concatenate([W_t[k] for k in range(K)], axis=1)
    CL = jnp.zeros((KD, KA), jnp.float32)
    CR = jnp.zeros((KD, ARW), jnp.float32)
    for k in range(K):
        for i in range(A + 1):
            CL = CL.at[k * DOUT:(k + 1) * DOUT, k * (A + 1) + i].set(W_l[k, i, :, 0])
            CR = CR.at[k * DOUT:(k + 1) * DOUT, k * (A + 1) + i].set(W_r[k, i, :, 0])

    perm = []
    for j in range(KD // 32):
        for t in range(L):
            perm.extend([32 * j + t, 32 * j + L + t])
    Wcatp = Wcat[:, jnp.array(perm, jnp.int32)]

    GB = 10
    WXB, AL, AR = pl.pallas_call(
        _dense_body,
        grid=(GB,),
        in_specs=[pl.BlockSpec((N // GB, DIN), lambda i: (i, 0)),
                  pl.BlockSpec((DIN, KD), lambda i: (0, 0)),
                  pl.BlockSpec((DIN, KD), lambda i: (0, 0)),
                  pl.BlockSpec((KD, KA), lambda i: (0, 0)),
                  pl.BlockSpec((KD, ARW), lambda i: (0, 0))],
        out_specs=[pl.BlockSpec((N // GB, KD), lambda i: (i, 0)),
                   pl.BlockSpec((N // GB, KA), lambda i: (i, 0)),
                   pl.BlockSpec((N // GB, ARW), lambda i: (i, 0))],
        out_shape=[jax.ShapeDtypeStruct((N, KD), jnp.bfloat16),
                   jax.ShapeDtypeStruct((N, KA), jnp.float32),
                   jax.ShapeDtypeStruct((N, ARW), jnp.float32)],
    )(x, Wcat, Wcatp, CL, CR)

    offs = pl.pallas_call(
        _offs_body,
        in_specs=[pl.BlockSpec((E // 128, 128), lambda: (0, 0))],
        out_specs=pl.BlockSpec((1, 128), lambda: (0, 0)),
        out_shape=jax.ShapeDtypeStruct((1, 128), jnp.int32),
    )(rows.reshape(E // 128, 128))

    fields = [rows, cols]
    for i in range(A):
        fields.append(lax.bitcast_convert_type(atten_values[i], jnp.int32))
    for k in range(K):
        fields.append(lax.bitcast_convert_type(support_values[k], jnp.int32))
    ed = jnp.stack(fields, axis=0)
    ed = jnp.concatenate([ed, jnp.zeros((NF, C), jnp.int32)], axis=1)

    AL_pad = jnp.concatenate([AL, jnp.zeros((NPAD - N, KA), jnp.float32)])

    mesh = plsc.VectorSubcoreMesh(core_axis_name="c", subcore_axis_name="s")
    out_pad = pl.kernel(
        _sc_body,
        out_type=jax.ShapeDtypeStruct((NPAD, DOUT), jnp.float32),
        mesh=mesh,
        compiler_params=pltpu.CompilerParams(needs_layout_passes=False,
                                             use_tc_tiling_on_sc=False),
        scratch_types=[
            [pltpu.VMEM((C, ARW), jnp.float32)] * NB,
            pltpu.VMEM((NPT, KA), jnp.float32),
            pltpu.VMEM((1, 128), jnp.int32),
            [pltpu.VMEM((NF, C + L), jnp.int32)] * NB,
            [pltpu.VMEM((C, KD), jnp.bfloat16)] * NB,
            pltpu.VMEM((K, C), jnp.float32),
            pltpu.VMEM((NPT, ACC_W), jnp.float32),
            [pltpu.SemaphoreType.DMA] * NB,
            [pltpu.SemaphoreType.DMA] * NB,
            [pltpu.SemaphoreType.DMA] * NB,
        ],
    )(WXB, ed, AL_pad, AR, offs)

    return out_pad[:N]
